# Initial kernel scaffold; baseline (speedup 1.0000x reference)
import functools

import jax
import jax.numpy as jnp
from jax import lax
from jax.experimental import pallas as pl
from jax.experimental.pallas import tpu as pltpu
from jax.experimental.pallas import tpu_sc as plsc

NS = 16
NCORE = 2
BATCH = 128
CHUNK = 8
ROW_BLOCK = 2000


@functools.lru_cache(maxsize=None)
def _make_spmm(n, nbat):
    acc_rows = n + NS
    span = acc_rows // NS
    bpt = nbat // NS
    nch = bpt // CHUNK
    assert bpt % CHUNK == 0 and acc_rows % NS == 0
    zq = zr = None
    for cand in (6, 3, 2, 1):
        if span % cand == 0 and span // cand <= 1024:
            zq, zr = cand, span // cand
            break
    assert zr is not None

    mesh = plsc.VectorSubcoreMesh(core_axis_name="c", subcore_axis_name="s")

    def body(u_hbm, comb_hbm, out_hbm, acc, idx_v, rows_v, zero_v, gsem, ssem):
        cid = lax.axis_index("c")
        sid = lax.axis_index("s")

        def _zb(i, c):
            zero_v[i, pl.ds(0, 16)] = jnp.zeros((16,), jnp.float32)
            zero_v[i, pl.ds(16, 16)] = jnp.zeros((16,), jnp.float32)
            return c
        lax.fori_loop(0, zr, _zb, 0)
        z0 = sid * span
        for q in range(zq):
            pltpu.sync_copy(zero_v, acc.at[pl.ds(z0 + q * zr, zr)])
        plsc.subcore_barrier()

        base = sid * bpt

        def chunk_body(k, c):
            cur = lax.rem(k, 2)
            prev = 1 - cur

            @pl.when(k > 0)
            def _():
                for j in range(CHUNK):
                    pltpu.make_async_copy(
                        rows_v.at[prev, j], acc.at[idx_v.at[prev, j, 1]], ssem
                    ).wait()

            pltpu.sync_copy(
                comb_hbm.at[cid, pl.ds(base + k * CHUNK, CHUNK)], idx_v.at[cur]
            )

            ds_ = [
                pltpu.async_copy(
                    u_hbm.at[idx_v.at[cur, j, 0]], rows_v.at[cur, j], gsem
                )
                for j in range(CHUNK)
            ]
            for d in ds_:
                d.wait()

            for j in range(CHUNK):
                pltpu.async_copy(
                    rows_v.at[cur, j], acc.at[idx_v.at[cur, j, 1]], ssem,
                    add=True,
                )
            return c

        lax.fori_loop(0, nch, chunk_body, 0)

        last = (nch - 1) % 2
        for j in range(CHUNK):
            pltpu.make_async_copy(
                rows_v.at[last, j], acc.at[idx_v.at[last, j, 1]], ssem
            ).wait()
        plsc.subcore_barrier()

        r0 = sid * span
        main = span - NS
        pltpu.sync_copy(
            acc.at[pl.ds(r0, main)], out_hbm.at[cid, pl.ds(r0, main)]
        )

        @pl.when(sid < NS - 1)
        def _():
            pltpu.sync_copy(
                acc.at[pl.ds(r0 + main, NS)],
                out_hbm.at[cid, pl.ds(r0 + main, NS)],
            )

    return pl.kernel(
        body,
        out_type=jax.ShapeDtypeStruct((NCORE, n, 32), jnp.float32),
        mesh=mesh,
        scratch_types=[
            pltpu.VMEM_SHARED((acc_rows, 32), jnp.float32),
            pltpu.VMEM((2, CHUNK, 2, BATCH), jnp.int32),
            pltpu.VMEM((2, CHUNK, BATCH, 32), jnp.float32),
            pltpu.VMEM((zr, 32), jnp.float32),
            pltpu.SemaphoreType.DMA,
            pltpu.SemaphoreType.DMA,
        ],
    )


def _full(shape):
    return pl.BlockSpec(shape, lambda i: (0,) * len(shape))


def _init_body(feat, wiT, bi, w1loT, w1hiT, b1lo, b1hi, h_out, u3_out):
    h = feat[...] @ wiT[...] + bi[...]
    h_out[...] = h
    u3_out[0] = jnp.maximum(h @ w1loT[...] + b1lo[...], 0.0)
    u3_out[1] = jnp.maximum(h @ w1hiT[...] + b1hi[...], 0.0)


def _upd_body(a3, h, GrloT, GrhiT, GzloT, GzhiT, GnloT, GnhiT,
              HrT, HzT, HnT, br, bz, bni, bnh,
              w1loT, w1hiT, b1lo, b1hi, h_out, u3_out):
    alo = a3[0]
    ahi = a3[1]
    hh = h[...]
    gr = alo @ GrloT[...] + ahi @ GrhiT[...] + hh @ HrT[...] + br[...]
    gz = alo @ GzloT[...] + ahi @ GzhiT[...] + hh @ HzT[...] + bz[...]
    gni = alo @ GnloT[...] + ahi @ GnhiT[...] + bni[...]
    gnh = hh @ HnT[...] + bnh[...]
    r = jax.nn.sigmoid(gr)
    z = jax.nn.sigmoid(gz)
    nn = jnp.tanh(gni + r * gnh)
    hn = (1.0 - z) * nn + z * hh
    h_out[...] = hn
    u3_out[0] = jnp.maximum(hn @ w1loT[...] + b1lo[...], 0.0)
    u3_out[1] = jnp.maximum(hn @ w1hiT[...] + b1hi[...], 0.0)


def _cls_body(h, w1T, b1, w2T, b2, out):
    t = jnp.maximum(h[...] @ w1T[...] + b1[...], 0.0)
    out[...] = t @ w2T[...] + b2[...]


@functools.lru_cache(maxsize=None)
def _make_tc(n, dim, dfeat):
    nb = n // ROW_BLOCK
    fB = ROW_BLOCK

    def row_spec(d):
        return pl.BlockSpec((fB, d), lambda i: (i, 0))

    def tri_spec():
        return pl.BlockSpec((2, fB, 32), lambda i: (0, i, 0))

    h_shape = jax.ShapeDtypeStruct((n, dim), jnp.float32)
    u3_shape = jax.ShapeDtypeStruct((2, n, 32), jnp.float32)

    init = pl.pallas_call(
        _init_body,
        grid=(nb,),
        in_specs=[
            row_spec(dfeat),
            _full((dfeat, dim)), _full((1, dim)),
            _full((dim, 32)), _full((dim, 32)),
            _full((1, 32)), _full((1, 32)),
        ],
        out_specs=[row_spec(dim), tri_spec()],
        out_shape=[h_shape, u3_shape],
    )

    upd = pl.pallas_call(
        _upd_body,
        grid=(nb,),
        in_specs=[
            tri_spec(), row_spec(dim),
            _full((32, dim)), _full((32, dim)),
            _full((32, dim)), _full((32, dim)),
            _full((32, dim)), _full((32, dim)),
            _full((dim, dim)), _full((dim, dim)), _full((dim, dim)),
            _full((1, dim)), _full((1, dim)), _full((1, dim)), _full((1, dim)),
            _full((dim, 32)), _full((dim, 32)),
            _full((1, 32)), _full((1, 32)),
        ],
        out_specs=[row_spec(dim), tri_spec()],
        out_shape=[h_shape, u3_shape],
    )

    return init, upd


@functools.lru_cache(maxsize=None)
def _make_cls(n, dim, dcls):
    nb = n // ROW_BLOCK

    return pl.pallas_call(
        _cls_body,
        grid=(nb,),
        in_specs=[
            pl.BlockSpec((ROW_BLOCK, dim), lambda i: (i, 0)),
            _full((dim, dcls)), _full((1, dcls)),
            _full((dcls, 1)), _full((1, 1)),
        ],
        out_specs=pl.BlockSpec((ROW_BLOCK, 1), lambda i: (i, 0)),
        out_shape=jax.ShapeDtypeStruct((n, 1), jnp.float32),
    )


def _fold_gru(dim, dhalf, w2, b2, wih):
    A_lo = jnp.zeros((dim, 32), jnp.float32)
    A_lo = A_lo.at[:, :dhalf].set(w2[:, :dhalf]).at[:, dhalf].set(b2)
    A_hi = jnp.zeros((dim, 32), jnp.float32)
    A_hi = A_hi.at[:, :dhalf].set(w2[:, dhalf:])
    G_lo = wih @ A_lo
    G_hi = wih @ A_hi
    out = []
    for g in range(3):
        out.append(G_lo[g * dim:(g + 1) * dim].T)
        out.append(G_hi[g * dim:(g + 1) * dim].T)
    return out


def _fold_next_mlp(dim, dhalf, w1, b1):
    W1lo = jnp.zeros((32, dim), jnp.float32).at[:dhalf].set(w1[:dhalf])
    b1lo = jnp.zeros((32,), jnp.float32).at[:dhalf].set(b1[:dhalf])
    b1lo = b1lo.at[dhalf].set(1.0)
    W1hi = jnp.zeros((32, dim), jnp.float32).at[:dhalf].set(w1[dhalf:])
    b1hi = jnp.zeros((32,), jnp.float32).at[:dhalf].set(b1[dhalf:])
    return W1lo.T, W1hi.T, b1lo.reshape(1, 32), b1hi.reshape(1, 32)


def _gru_rest(dim, whh, bih, bhh):
    HrT = whh[0:dim].T
    HzT = whh[dim:2 * dim].T
    HnT = whh[2 * dim:].T
    br = (bih[0:dim] + bhh[0:dim]).reshape(1, dim)
    bz = (bih[dim:2 * dim] + bhh[dim:2 * dim]).reshape(1, dim)
    bni = bih[2 * dim:].reshape(1, dim)
    bnh = bhh[2 * dim:].reshape(1, dim)
    return HrT, HzT, HnT, br, bz, bni, bnh


def _build_comb(n, e_pad, src, dst):
    e = src.shape[0]
    pad = e_pad - e
    ar = jnp.arange(pad, dtype=jnp.int32)
    s = jnp.concatenate([src, ar % 64]).reshape(-1, BATCH)
    d = jnp.concatenate([dst, n + (ar % NS)]).reshape(-1, BATCH)
    c0 = jnp.stack([s, d], 1)
    c1 = jnp.stack([s + n, d], 1)
    return jnp.stack([c0, c1], 0)


def kernel(features, edge_index, W_init, b_init,
           fm_w1, fm_b1, fm_w2, fm_b2,
           bm_w1, bm_b1, bm_w2, bm_b2,
           fg_wih, fg_whh, fg_bih, fg_bhh,
           bg_wih, bg_whh, bg_bih, bg_bhh,
           cl_w1, cl_b1, cl_w2, cl_b2, n_rounds=20):
    n, dfeat = features.shape
    e = edge_index.shape[1]
    dim = W_init.shape[0]
    dhalf = fm_w1.shape[0] // 2
    dcls = cl_w1.shape[0]

    group = NS * CHUNK * BATCH
    e_pad = ((e + group - 1) // group) * group
    nbat = e_pad // BATCH

    row = edge_index[0]
    col = edge_index[1]
    comb_f = _build_comb(n, e_pad, col, row)
    comb_b = _build_comb(n, e_pad, row, col)

    f_upd = (
        tuple(_fold_gru(dim, dhalf, fm_w2, fm_b2, fg_wih))
        + _gru_rest(dim, fg_whh, fg_bih, fg_bhh)
        + _fold_next_mlp(dim, dhalf, bm_w1, bm_b1)
    )
    b_upd = (
        tuple(_fold_gru(dim, dhalf, bm_w2, bm_b2, bg_wih))
        + _gru_rest(dim, bg_whh, bg_bih, bg_bhh)
        + _fold_next_mlp(dim, dhalf, fm_w1, fm_b1)
    )
    init_w = _fold_next_mlp(dim, dhalf, fm_w1, fm_b1)

    spmm = _make_spmm(n, nbat)
    init, upd = _make_tc(n, dim, dfeat)
    cls = _make_cls(n, dim, dcls)

    h, u3 = init(features, W_init.T, b_init.reshape(1, dim), *init_w)

    def round_body(_, carry):
        h, u3 = carry
        agg = spmm(u3.reshape(2 * n, 32), comb_f)
        h, u3 = upd(agg, h, *f_upd)
        agg = spmm(u3.reshape(2 * n, 32), comb_b)
        h, u3 = upd(agg, h, *b_upd)
        return h, u3

    h, u3 = lax.fori_loop(0, n_rounds, round_body, (h, u3))

    return cls(h, cl_w1.T, cl_b1.reshape(1, dcls), cl_w2.T,
               jnp.float32(cl_b2).reshape(1, 1))


# speedup vs baseline: 5.4207x; 5.4207x over previous
import functools

import jax
import jax.numpy as jnp
from jax import lax
from jax.experimental import pallas as pl
from jax.experimental.pallas import tpu as pltpu
from jax.experimental.pallas import tpu_sc as plsc

NS = 16
NCORE = 2
BATCH = 128
CHUNK = 2
ROW_BLOCK = 2000

_HI = jax.lax.Precision.HIGHEST


def _mm(a, b):
    return jnp.matmul(a, b, precision=_HI)


@functools.lru_cache(maxsize=None)
def _make_spmm(n, nbat):
    acc_rows = ((n + NS * 8 - 1) // (NS * 8) + 1) * NS * 8
    garbage = acc_rows - n
    span = acc_rows // NS
    bpt = nbat // NS
    nch = bpt // CHUNK
    assert bpt % CHUNK == 0 and span % 8 == 0 and garbage % 8 == 0
    nz128, zrem = divmod(span, BATCH)
    assert zrem % 8 == 0

    mesh = plsc.VectorSubcoreMesh(
        core_axis_name="c", subcore_axis_name="s",
        num_cores=NCORE, num_subcores=NS,
    )

    def body(u_hbm, comb_hbm, out_hbm, acc, idx_v, rows_v, gsem, ssem):
        cid = lax.axis_index("c")
        sid = lax.axis_index("s")

        def _zb(i, c):
            rows_v[0, 0, i, pl.ds(0, 16)] = jnp.zeros((16,), jnp.float32)
            rows_v[0, 0, i, pl.ds(16, 16)] = jnp.zeros((16,), jnp.float32)
            return c
        lax.fori_loop(0, BATCH, _zb, 0)
        z0 = pl.multiple_of(sid * span, 8)

        def _zc(q, c):
            pltpu.sync_copy(
                rows_v.at[0, 0],
                acc.at[pl.ds(pl.multiple_of(z0 + q * BATCH, 8), BATCH)],
            )
            return c
        lax.fori_loop(0, nz128, _zc, 0)
        if zrem:
            pltpu.sync_copy(
                rows_v.at[0, 0, pl.ds(0, zrem)],
                acc.at[pl.ds(pl.multiple_of(z0 + nz128 * BATCH, 8), zrem)],
            )
        plsc.subcore_barrier()

        base = sid * bpt

        def chunk_body(k, c):
            cur = lax.rem(k, 2)
            prev = 1 - cur

            @pl.when(k > 0)
            def _():
                for j in range(CHUNK):
                    pltpu.make_async_copy(
                        rows_v.at[prev, j], acc.at[idx_v.at[prev, j, 1]], ssem
                    ).wait()

            pltpu.sync_copy(
                comb_hbm.at[cid, pl.ds(base + k * CHUNK, CHUNK)], idx_v.at[cur]
            )

            ds_ = [
                pltpu.async_copy(
                    u_hbm.at[idx_v.at[cur, j, 0]], rows_v.at[cur, j], gsem
                )
                for j in range(CHUNK)
            ]
            for d in ds_:
                d.wait()

            for j in range(CHUNK):
                pltpu.async_copy(
                    rows_v.at[cur, j], acc.at[idx_v.at[cur, j, 1]], ssem,
                    add=True,
                )
            return c

        lax.fori_loop(0, nch, chunk_body, 0)

        last = (nch - 1) % 2
        for j in range(CHUNK):
            pltpu.make_async_copy(
                rows_v.at[last, j], acc.at[idx_v.at[last, j, 1]], ssem
            ).wait()
        plsc.subcore_barrier()

        r0 = pl.multiple_of(sid * span, 8)

        @pl.when(sid < NS - 1)
        def _():
            pltpu.sync_copy(
                acc.at[pl.ds(r0, span)], out_hbm.at[cid, pl.ds(r0, span)]
            )

        @pl.when(sid == NS - 1)
        def _():
            pltpu.sync_copy(
                acc.at[pl.ds(r0, span - garbage)],
                out_hbm.at[cid, pl.ds(r0, span - garbage)],
            )

    return pl.kernel(
        body,
        out_type=jax.ShapeDtypeStruct((NCORE, n, 32), jnp.float32),
        mesh=mesh,
        compiler_params=pltpu.CompilerParams(use_tc_tiling_on_sc=False),
        scratch_types=[
            pltpu.VMEM_SHARED((acc_rows, 32), jnp.float32),
            pltpu.VMEM((2, CHUNK, 2, BATCH), jnp.int32),
            pltpu.VMEM((2, CHUNK, BATCH, 32), jnp.float32),
            pltpu.SemaphoreType.DMA,
            pltpu.SemaphoreType.DMA,
        ],
    )


def _full(shape):
    return pl.BlockSpec(shape, lambda i: (0,) * len(shape))


def _init_body(feat, wiT, bi, w1loT, w1hiT, b1lo, b1hi, h_out, u3_out):
    h = _mm(feat[...], wiT[...]) + bi[...]
    h_out[...] = h
    u3_out[0] = jnp.maximum(_mm(h, w1loT[...]) + b1lo[...], 0.0)
    u3_out[1] = jnp.maximum(_mm(h, w1hiT[...]) + b1hi[...], 0.0)


def _upd_body(a3, h, GrloT, GrhiT, GzloT, GzhiT, GnloT, GnhiT,
              HrT, HzT, HnT, br, bz, bni, bnh,
              w1loT, w1hiT, b1lo, b1hi, h_out, u3_out):
    alo = a3[0]
    ahi = a3[1]
    hh = h[...]
    gr = _mm(alo, GrloT[...]) + _mm(ahi, GrhiT[...]) + _mm(hh, HrT[...]) + br[...]
    gz = _mm(alo, GzloT[...]) + _mm(ahi, GzhiT[...]) + _mm(hh, HzT[...]) + bz[...]
    gni = _mm(alo, GnloT[...]) + _mm(ahi, GnhiT[...]) + bni[...]
    gnh = _mm(hh, HnT[...]) + bnh[...]
    r = jax.nn.sigmoid(gr)
    z = jax.nn.sigmoid(gz)
    nn = jnp.tanh(gni + r * gnh)
    hn = (1.0 - z) * nn + z * hh
    h_out[...] = hn
    u3_out[0] = jnp.maximum(_mm(hn, w1loT[...]) + b1lo[...], 0.0)
    u3_out[1] = jnp.maximum(_mm(hn, w1hiT[...]) + b1hi[...], 0.0)


def _cls_body(h, w1T, b1, w2T, b2, out):
    t = jnp.maximum(_mm(h[...], w1T[...]) + b1[...], 0.0)
    out[...] = _mm(t, w2T[...]) + b2[...]


@functools.lru_cache(maxsize=None)
def _make_tc(n, dim, dfeat):
    nb = n // ROW_BLOCK
    fB = ROW_BLOCK

    def row_spec(d):
        return pl.BlockSpec((fB, d), lambda i: (i, 0))

    def tri_spec():
        return pl.BlockSpec((2, fB, 32), lambda i: (0, i, 0))

    h_shape = jax.ShapeDtypeStruct((n, dim), jnp.float32)
    u3_shape = jax.ShapeDtypeStruct((2, n, 32), jnp.float32)

    init = pl.pallas_call(
        _init_body,
        grid=(nb,),
        in_specs=[
            row_spec(dfeat),
            _full((dfeat, dim)), _full((1, dim)),
            _full((dim, 32)), _full((dim, 32)),
            _full((1, 32)), _full((1, 32)),
        ],
        out_specs=[row_spec(dim), tri_spec()],
        out_shape=[h_shape, u3_shape],
    )

    upd = pl.pallas_call(
        _upd_body,
        grid=(nb,),
        in_specs=[
            tri_spec(), row_spec(dim),
            _full((32, dim)), _full((32, dim)),
            _full((32, dim)), _full((32, dim)),
            _full((32, dim)), _full((32, dim)),
            _full((dim, dim)), _full((dim, dim)), _full((dim, dim)),
            _full((1, dim)), _full((1, dim)), _full((1, dim)), _full((1, dim)),
            _full((dim, 32)), _full((dim, 32)),
            _full((1, 32)), _full((1, 32)),
        ],
        out_specs=[row_spec(dim), tri_spec()],
        out_shape=[h_shape, u3_shape],
    )

    return init, upd


@functools.lru_cache(maxsize=None)
def _make_cls(n, dim, dcls):
    nb = n // ROW_BLOCK

    return pl.pallas_call(
        _cls_body,
        grid=(nb,),
        in_specs=[
            pl.BlockSpec((ROW_BLOCK, dim), lambda i: (i, 0)),
            _full((dim, dcls)), _full((1, dcls)),
            _full((dcls, 1)), _full((1, 1)),
        ],
        out_specs=pl.BlockSpec((ROW_BLOCK, 1), lambda i: (i, 0)),
        out_shape=jax.ShapeDtypeStruct((n, 1), jnp.float32),
    )


def _fold_gru(dim, dhalf, w2, b2, wih):
    A_lo = jnp.zeros((dim, 32), jnp.float32)
    A_lo = A_lo.at[:, :dhalf].set(w2[:, :dhalf]).at[:, dhalf].set(b2)
    A_hi = jnp.zeros((dim, 32), jnp.float32)
    A_hi = A_hi.at[:, :dhalf].set(w2[:, dhalf:])
    G_lo = _mm(wih, A_lo)
    G_hi = _mm(wih, A_hi)
    out = []
    for g in range(3):
        out.append(G_lo[g * dim:(g + 1) * dim].T)
        out.append(G_hi[g * dim:(g + 1) * dim].T)
    return out


def _fold_next_mlp(dim, dhalf, w1, b1):
    W1lo = jnp.zeros((32, dim), jnp.float32).at[:dhalf].set(w1[:dhalf])
    b1lo = jnp.zeros((32,), jnp.float32).at[:dhalf].set(b1[:dhalf])
    b1lo = b1lo.at[dhalf].set(1.0)
    W1hi = jnp.zeros((32, dim), jnp.float32).at[:dhalf].set(w1[dhalf:])
    b1hi = jnp.zeros((32,), jnp.float32).at[:dhalf].set(b1[dhalf:])
    return W1lo.T, W1hi.T, b1lo.reshape(1, 32), b1hi.reshape(1, 32)


def _gru_rest(dim, whh, bih, bhh):
    HrT = whh[0:dim].T
    HzT = whh[dim:2 * dim].T
    HnT = whh[2 * dim:].T
    br = (bih[0:dim] + bhh[0:dim]).reshape(1, dim)
    bz = (bih[dim:2 * dim] + bhh[dim:2 * dim]).reshape(1, dim)
    bni = bih[2 * dim:].reshape(1, dim)
    bnh = bhh[2 * dim:].reshape(1, dim)
    return HrT, HzT, HnT, br, bz, bni, bnh


def _build_comb(n, e_pad, src, dst, gar):
    e = src.shape[0]
    pad = e_pad - e
    ar = jnp.arange(pad, dtype=jnp.int32)
    s = jnp.concatenate([src, ar % 64]).reshape(-1, BATCH)
    d = jnp.concatenate([dst, n + (ar % gar)]).reshape(-1, BATCH)
    c0 = jnp.stack([s, d], 1)
    c1 = jnp.stack([s + n, d], 1)
    return jnp.stack([c0, c1], 0)


def kernel(features, edge_index, W_init, b_init,
           fm_w1, fm_b1, fm_w2, fm_b2,
           bm_w1, bm_b1, bm_w2, bm_b2,
           fg_wih, fg_whh, fg_bih, fg_bhh,
           bg_wih, bg_whh, bg_bih, bg_bhh,
           cl_w1, cl_b1, cl_w2, cl_b2, n_rounds=20):
    n, dfeat = features.shape
    e = edge_index.shape[1]
    dim = W_init.shape[0]
    dhalf = fm_w1.shape[0] // 2
    dcls = cl_w1.shape[0]

    group = NS * CHUNK * BATCH
    e_pad = ((e + group - 1) // group) * group
    nbat = e_pad // BATCH

    gar = ((n + NS * 8 - 1) // (NS * 8) + 1) * NS * 8 - n
    row = edge_index[0]
    col = edge_index[1]
    comb_f = _build_comb(n, e_pad, col, row, gar)
    comb_b = _build_comb(n, e_pad, row, col, gar)

    f_upd = (
        tuple(_fold_gru(dim, dhalf, fm_w2, fm_b2, fg_wih))
        + _gru_rest(dim, fg_whh, fg_bih, fg_bhh)
        + _fold_next_mlp(dim, dhalf, bm_w1, bm_b1)
    )
    b_upd = (
        tuple(_fold_gru(dim, dhalf, bm_w2, bm_b2, bg_wih))
        + _gru_rest(dim, bg_whh, bg_bih, bg_bhh)
        + _fold_next_mlp(dim, dhalf, fm_w1, fm_b1)
    )
    init_w = _fold_next_mlp(dim, dhalf, fm_w1, fm_b1)

    spmm = _make_spmm(n, nbat)
    init, upd = _make_tc(n, dim, dfeat)
    cls = _make_cls(n, dim, dcls)

    h, u3 = init(features, W_init.T, b_init.reshape(1, dim), *init_w)

    def round_body(_, carry):
        h, u3 = carry
        agg = spmm(u3.reshape(2 * n, 32), comb_f)
        h, u3 = upd(agg, h, *f_upd)
        agg = spmm(u3.reshape(2 * n, 32), comb_b)
        h, u3 = upd(agg, h, *b_upd)
        return h, u3

    h, u3 = lax.fori_loop(0, n_rounds, round_body, (h, u3))

    return cls(h, cl_w1.T, cl_b1.reshape(1, dcls), cl_w2.T,
               cl_b2.reshape(1, 1))
